# Initial kernel scaffold; baseline (speedup 1.0000x reference)
import functools

import jax
import jax.numpy as jnp
from jax import lax
from jax.experimental import pallas as pl
from jax.experimental.pallas import tpu as pltpu
from jax.experimental.pallas import tpu_sc as plsc

B = 16384
F = 26
V = 1001
D = 128
ND = 13

NC = 2
NS = 16
NW = NC * NS
BW = B // NW
CH = 128
NCH = BW // CH
IDX_ROWS = F * B // CH


def _sc_gather_body(idx_hbm, so_hbm, fo_hbm, emb_out, fo_out,
                    idx_v, rows_v, fo_v, acc_v, sem_g, sem_s):
    wid = lax.axis_index("s") * NC + lax.axis_index("c")
    pltpu.sync_copy(fo_hbm, fo_v)
    for j in range(BW // 16):
        acc_v[pl.ds(j * 16, 16)] = jnp.zeros((16,), jnp.float32)

    row_base = wid * NCH

    def field_body(f, _):
        r0 = f * (B // CH) + row_base
        pltpu.sync_copy(idx_hbm.at[pl.ds(r0, NCH)], idx_v)

        @pl.when(f > 0)
        def _():
            pltpu.make_async_copy(rows_v, emb_out.at[pl.ds(0, NCH)], sem_s).wait()

        for c in range(NCH):
            pltpu.async_copy(so_hbm.at[idx_v.at[c]], rows_v.at[c], sem_g)
        for c in range(NCH):
            pltpu.make_async_copy(so_hbm.at[idx_v.at[c]], rows_v.at[c], sem_g).wait()

        for c in range(NCH):
            for j in range(CH // 16):
                iv = idx_v[c, pl.ds(j * 16, 16)]
                sl = pl.ds(c * CH + j * 16, 16)
                acc_v[sl] = acc_v[sl] + plsc.load_gather(fo_v, [iv])

        pltpu.async_copy(rows_v, emb_out.at[pl.ds(r0, NCH)], sem_s)
        return 0

    lax.fori_loop(0, F, field_body, 0)
    pltpu.make_async_copy(rows_v, emb_out.at[pl.ds(0, NCH)], sem_s).wait()
    pltpu.sync_copy(acc_v, fo_out.at[pl.ds(wid * BW, BW)])


def _sc_gather(idx2d, so_flat, fo_flat):
    mesh = plsc.VectorSubcoreMesh(core_axis_name="c", subcore_axis_name="s")
    return pl.kernel(
        _sc_gather_body,
        out_type=(
            jax.ShapeDtypeStruct((IDX_ROWS, CH, D), jnp.float32),
            jax.ShapeDtypeStruct((B,), jnp.float32),
        ),
        mesh=mesh,
        scratch_types=[
            pltpu.VMEM((NCH, CH), jnp.int32),
            pltpu.VMEM((NCH, CH, D), jnp.float32),
            pltpu.VMEM((F * V,), jnp.float32),
            pltpu.VMEM((BW,), jnp.float32),
            pltpu.SemaphoreType.DMA,
            pltpu.SemaphoreType.DMA,
        ],
    )(idx2d, so_flat, fo_flat)


BT = 512


def _tc_body(num_ref, emb_ref, fo_ref, wnum_ref, bnum_ref,
             w0a_ref, w0b_ref, b0_ref, w1_ref, b1_ref, w2_ref, b2_ref,
             wfm_ref, wh_ref, bout_ref, out_ref):
    num = num_ref[...]
    e = emb_ref[...]

    s = jnp.sum(e, axis=0)
    sumsq = jnp.sum(s * s, axis=1, keepdims=True)
    sqsum = jnp.sum(jnp.sum(e * e, axis=0), axis=1, keepdims=True)
    fm2 = 0.5 * (sumsq - sqsum)

    fm1 = jnp.dot(num, wnum_ref[...], preferred_element_type=jnp.float32)
    fm1 = fm1 + bnum_ref[...] + fo_ref[...]
    fm = fm1 + fm2

    h = jnp.dot(num, w0a_ref[...], preferred_element_type=jnp.float32)
    for f in range(F):
        h = h + jnp.dot(e[f], w0b_ref[f], preferred_element_type=jnp.float32)
    h = jnp.maximum(h + b0_ref[...], 0.0)
    h = jnp.maximum(jnp.dot(h, w1_ref[...], preferred_element_type=jnp.float32)
                    + b1_ref[...], 0.0)
    h = jnp.maximum(jnp.dot(h, w2_ref[...], preferred_element_type=jnp.float32)
                    + b2_ref[...], 0.0)

    total = fm * wfm_ref[0, 0] + jnp.dot(h, wh_ref[...],
                                         preferred_element_type=jnp.float32)
    total = total + bout_ref[...]
    out_ref[...] = jax.nn.sigmoid(total)


def _tc_mlp(numeric, emb3, fo_sum2, W_num, b_num, W0a, W0b, b0, W1, b1, W2, b2,
            Wfm, Wh, bout):
    grid = (B // BT,)

    def full(shape):
        return pl.BlockSpec(shape, lambda *_: tuple(0 for _ in shape))

    return pl.pallas_call(
        _tc_body,
        grid=grid,
        in_specs=[
            pl.BlockSpec((BT, ND), lambda i: (i, 0)),
            pl.BlockSpec((F, BT, D), lambda i: (0, i, 0)),
            pl.BlockSpec((BT, 1), lambda i: (i, 0)),
            full((ND, 1)),
            full((1, 1)),
            full((ND, 1024)),
            full((F, D, 1024)),
            full((1, 1024)),
            full((1024, 512)),
            full((1, 512)),
            full((512, 256)),
            full((1, 256)),
            full((1, 1)),
            full((256, 1)),
            full((1, 1)),
        ],
        out_specs=pl.BlockSpec((BT, 1), lambda i: (i, 0)),
        out_shape=jax.ShapeDtypeStruct((B, 1), jnp.float32),
    )(numeric, emb3, fo_sum2, W_num, b_num, W0a, W0b, b0, W1, b1, W2, b2,
      Wfm, Wh, bout)


@jax.jit
def kernel(numeric, categorical, W_num, b_num, fo_tables, so_tables,
           W0, b0, W1, b1, W2, b2, Wout, bout):
    offs = (jnp.arange(F, dtype=jnp.int32) * V)[:, None]
    idx2d = (categorical.T + offs).reshape(IDX_ROWS, CH)
    so_flat = so_tables.reshape(F * V, D)
    fo_flat = fo_tables.reshape(F * V)

    emb, fo_sum = _sc_gather(idx2d, so_flat, fo_flat)
    emb3 = emb.reshape(F, B, D)
    fo_sum2 = fo_sum.reshape(B, 1)

    W0a = W0[:ND]
    W0b = W0[ND:].reshape(F, D, 1024)
    out = _tc_mlp(numeric, emb3, fo_sum2, W_num, b_num.reshape(1, 1),
                  W0a, W0b, b0.reshape(1, -1), W1, b1.reshape(1, -1),
                  W2, b2.reshape(1, -1), Wout[0:1], Wout[1:],
                  bout.reshape(1, 1))
    return out[:, 0]


# speedup vs baseline: 25.9972x; 25.9972x over previous
import functools

import jax
import jax.numpy as jnp
from jax import lax
from jax.experimental import pallas as pl
from jax.experimental.pallas import tpu as pltpu
from jax.experimental.pallas import tpu_sc as plsc

B = 16384
F = 26
V = 1001
D = 128
ND = 13

NC = 2
NS = 16
NW = NC * NS
BW = B // NW
CH = 128
NCH = BW // CH
IDX_ROWS = F * B // CH


def _sc_gather_body(idx_hbm, so_hbm, fo_hbm, emb_out, fo_out,
                    idx_v, rows_v, fo_v, acc_v, sem_g, sem_s):
    wid = lax.axis_index("s") * NC + lax.axis_index("c")
    pltpu.sync_copy(fo_hbm, fo_v)
    for j in range(BW // 16):
        acc_v[pl.ds(j * 16, 16)] = jnp.zeros((16,), jnp.float32)

    row_base = wid * NCH

    def field_body(f, _):
        r0 = f * (B // CH) + row_base
        pltpu.sync_copy(idx_hbm.at[pl.ds(r0, NCH)], idx_v)

        @pl.when(f > 0)
        def _():
            pltpu.make_async_copy(rows_v, emb_out.at[pl.ds(0, NCH)], sem_s).wait()

        for c in range(NCH):
            pltpu.async_copy(so_hbm.at[idx_v.at[c]], rows_v.at[c], sem_g)
        for c in range(NCH):
            pltpu.make_async_copy(so_hbm.at[idx_v.at[c]], rows_v.at[c], sem_g).wait()

        for c in range(NCH):
            for j in range(CH // 16):
                iv = idx_v[c, pl.ds(j * 16, 16)]
                sl = pl.ds(c * CH + j * 16, 16)
                acc_v[sl] = acc_v[sl] + plsc.load_gather(fo_v, [iv])

        pltpu.async_copy(rows_v, emb_out.at[pl.ds(r0, NCH)], sem_s)
        return 0

    lax.fori_loop(0, F, field_body, 0)
    pltpu.make_async_copy(rows_v, emb_out.at[pl.ds(0, NCH)], sem_s).wait()
    pltpu.sync_copy(acc_v, fo_out.at[pl.ds(wid * BW, BW)])


def _sc_gather(idx2d, so_flat, fo_flat):
    mesh = plsc.VectorSubcoreMesh(core_axis_name="c", subcore_axis_name="s")
    return pl.kernel(
        _sc_gather_body,
        out_type=(
            jax.ShapeDtypeStruct((IDX_ROWS, CH, D), jnp.float32),
            jax.ShapeDtypeStruct((B,), jnp.float32),
        ),
        mesh=mesh,
        compiler_params=pltpu.CompilerParams(needs_layout_passes=False),
        scratch_types=[
            pltpu.VMEM((NCH, CH), jnp.int32),
            pltpu.VMEM((NCH, CH, D), jnp.float32),
            pltpu.VMEM((F * V,), jnp.float32),
            pltpu.VMEM((BW,), jnp.float32),
            pltpu.SemaphoreType.DMA,
            pltpu.SemaphoreType.DMA,
        ],
    )(idx2d, so_flat, fo_flat)


BT = 512


def _tc_body(num_ref, emb_ref, fo_ref, wnum_ref, bnum_ref,
             w0a_ref, w0b_ref, b0_ref, w1_ref, b1_ref, w2_ref, b2_ref,
             wfm_ref, wh_ref, bout_ref, out_ref):
    num = num_ref[...]
    e = emb_ref[...]

    s = jnp.sum(e, axis=0)
    sumsq = jnp.sum(s * s, axis=1, keepdims=True)
    sqsum = jnp.sum(jnp.sum(e * e, axis=0), axis=1, keepdims=True)
    fm2 = 0.5 * (sumsq - sqsum)

    fm1 = jnp.dot(num, wnum_ref[...], preferred_element_type=jnp.float32)
    fm1 = fm1 + bnum_ref[...] + fo_ref[...]
    fm = fm1 + fm2

    h = jnp.dot(num, w0a_ref[...], preferred_element_type=jnp.float32)
    for f in range(F):
        h = h + jnp.dot(e[f], w0b_ref[f], preferred_element_type=jnp.float32)
    h = jnp.maximum(h + b0_ref[...], 0.0)
    h = jnp.maximum(jnp.dot(h, w1_ref[...], preferred_element_type=jnp.float32)
                    + b1_ref[...], 0.0)
    h = jnp.maximum(jnp.dot(h, w2_ref[...], preferred_element_type=jnp.float32)
                    + b2_ref[...], 0.0)

    total = fm * wfm_ref[0, 0] + jnp.dot(h, wh_ref[...],
                                         preferred_element_type=jnp.float32)
    total = total + bout_ref[...]
    out_ref[...] = jax.nn.sigmoid(total)


def _tc_mlp(numeric, emb3, fo_sum2, W_num, b_num, W0a, W0b, b0, W1, b1, W2, b2,
            Wfm, Wh, bout):
    grid = (B // BT,)

    def full(shape):
        return pl.BlockSpec(shape, lambda *_: tuple(0 for _ in shape))

    return pl.pallas_call(
        _tc_body,
        grid=grid,
        in_specs=[
            pl.BlockSpec((BT, ND), lambda i: (i, 0)),
            pl.BlockSpec((F, BT, D), lambda i: (0, i, 0)),
            pl.BlockSpec((BT, 1), lambda i: (i, 0)),
            full((ND, 1)),
            full((1, 1)),
            full((ND, 1024)),
            full((F, D, 1024)),
            full((1, 1024)),
            full((1024, 512)),
            full((1, 512)),
            full((512, 256)),
            full((1, 256)),
            full((1, 1)),
            full((256, 1)),
            full((1, 1)),
        ],
        out_specs=pl.BlockSpec((BT, 1), lambda i: (i, 0)),
        out_shape=jax.ShapeDtypeStruct((B, 1), jnp.float32),
    )(numeric, emb3, fo_sum2, W_num, b_num, W0a, W0b, b0, W1, b1, W2, b2,
      Wfm, Wh, bout)


@jax.jit
def kernel(numeric, categorical, W_num, b_num, fo_tables, so_tables,
           W0, b0, W1, b1, W2, b2, Wout, bout):
    offs = (jnp.arange(F, dtype=jnp.int32) * V)[:, None]
    idx2d = (categorical.T + offs).reshape(IDX_ROWS, CH)
    so_flat = so_tables.reshape(F * V, D)
    fo_flat = fo_tables.reshape(F * V)

    emb, fo_sum = _sc_gather(idx2d, so_flat, fo_flat)
    emb3 = emb.reshape(F, B, D)
    fo_sum2 = fo_sum.reshape(B, 1)

    W0a = W0[:ND]
    W0b = W0[ND:].reshape(F, D, 1024)
    out = _tc_mlp(numeric, emb3, fo_sum2, W_num, b_num.reshape(1, 1),
                  W0a, W0b, b0.reshape(1, -1), W1, b1.reshape(1, -1),
                  W2, b2.reshape(1, -1), Wout[0:1], Wout[1:],
                  bout.reshape(1, 1))
    return out[:, 0]


# speedup vs baseline: 32.3050x; 1.2426x over previous
import functools

import jax
import jax.numpy as jnp
from jax import lax
from jax.experimental import pallas as pl
from jax.experimental.pallas import tpu as pltpu
from jax.experimental.pallas import tpu_sc as plsc

B = 16384
F = 26
V = 1001
D = 128
ND = 13
H0 = 1024

NC = 2
NS = 16
NW = NC * NS
BW = B // NW
CH = 128
NCH = BW // CH
IDX_ROWS = F * B // CH


def _sc_gather_body(idx_hbm, so_hbm, fo_hbm, emb_out, fo_out,
                    idx_v, rows_v, fo_v, acc_v, sem_g, sem_s):
    wid = lax.axis_index("s") * NC + lax.axis_index("c")
    b0 = wid * BW
    pltpu.sync_copy(fo_hbm, fo_v)
    for j in range(BW // 16):
        acc_v[pl.ds(j * 16, 16)] = jnp.zeros((16,), jnp.float32)

    row_base = wid * NCH

    def field_body(f, _):
        r0 = f * (B // CH) + row_base
        pltpu.sync_copy(idx_hbm.at[pl.ds(r0, NCH)], idx_v)

        @pl.when(f > 0)
        def _():
            pltpu.make_async_copy(
                rows_v, emb_out.at[pl.ds(0, BW), pl.ds(0, D)], sem_s).wait()

        for c in range(NCH):
            pltpu.async_copy(so_hbm.at[idx_v.at[c]],
                             rows_v.at[pl.ds(c * CH, CH)], sem_g)
        for c in range(NCH):
            pltpu.make_async_copy(so_hbm.at[idx_v.at[c]],
                                  rows_v.at[pl.ds(c * CH, CH)], sem_g).wait()

        for c in range(NCH):
            for j in range(CH // 16):
                iv = idx_v[c, pl.ds(j * 16, 16)]
                sl = pl.ds(c * CH + j * 16, 16)
                acc_v[sl] = acc_v[sl] + plsc.load_gather(fo_v, [iv])

        pltpu.async_copy(
            rows_v, emb_out.at[pl.ds(b0, BW), pl.ds(f * D, D)], sem_s)
        return 0

    lax.fori_loop(0, F, field_body, 0)
    pltpu.make_async_copy(
        rows_v, emb_out.at[pl.ds(0, BW), pl.ds(0, D)], sem_s).wait()
    pltpu.sync_copy(acc_v, fo_out.at[pl.ds(b0, BW)])


def _sc_gather(idx2d, so_flat, fo_flat):
    mesh = plsc.VectorSubcoreMesh(core_axis_name="c", subcore_axis_name="s")
    return pl.kernel(
        _sc_gather_body,
        out_type=(
            jax.ShapeDtypeStruct((B, F * D), jnp.float32),
            jax.ShapeDtypeStruct((B,), jnp.float32),
        ),
        mesh=mesh,
        compiler_params=pltpu.CompilerParams(needs_layout_passes=False),
        scratch_types=[
            pltpu.VMEM((NCH, CH), jnp.int32),
            pltpu.VMEM((BW, D), jnp.float32),
            pltpu.VMEM((F * V,), jnp.float32),
            pltpu.VMEM((BW,), jnp.float32),
            pltpu.SemaphoreType.DMA,
            pltpu.SemaphoreType.DMA,
        ],
    )(idx2d, so_flat, fo_flat)


BT = 512


def _tc_body(num_ref, emb_ref, fo_ref, wnum_ref, bnum_ref,
             w0a_ref, w0b_ref, b0_ref, w1_ref, b1_ref, w2_ref, b2_ref,
             wfm_ref, wh_ref, bout_ref, out_ref):
    num = num_ref[...]
    flat = emb_ref[...]

    s = jnp.zeros((BT, D), jnp.float32)
    q = jnp.zeros((BT, D), jnp.float32)
    for f in range(F):
        x = flat[:, f * D:(f + 1) * D]
        s = s + x
        q = q + x * x
    sumsq = jnp.sum(s * s, axis=1, keepdims=True)
    sqsum = jnp.sum(q, axis=1, keepdims=True)
    fm2 = 0.5 * (sumsq - sqsum)

    fm1 = jnp.dot(num, wnum_ref[...], preferred_element_type=jnp.float32)
    fm1 = fm1 + bnum_ref[...] + fo_ref[...]
    fm = fm1 + fm2

    bf = jnp.bfloat16
    h = jnp.dot(num, w0a_ref[...], preferred_element_type=jnp.float32)
    h = h + jnp.dot(flat.astype(bf), w0b_ref[...],
                    preferred_element_type=jnp.float32)
    h = jnp.maximum(h + b0_ref[...], 0.0)
    h = jnp.maximum(jnp.dot(h.astype(bf), w1_ref[...],
                            preferred_element_type=jnp.float32)
                    + b1_ref[...], 0.0)
    h = jnp.maximum(jnp.dot(h.astype(bf), w2_ref[...],
                            preferred_element_type=jnp.float32)
                    + b2_ref[...], 0.0)

    total = fm * wfm_ref[0, 0] + jnp.dot(h, wh_ref[...],
                                         preferred_element_type=jnp.float32)
    total = total + bout_ref[...]
    out_ref[...] = jax.nn.sigmoid(total)


def _tc_mlp(numeric, emb2, fo_sum2, W_num, b_num, W0a, W0b, b0, W1, b1, W2, b2,
            Wfm, Wh, bout):
    grid = (B // BT,)

    def full(shape):
        return pl.BlockSpec(shape, lambda *_: tuple(0 for _ in shape))

    return pl.pallas_call(
        _tc_body,
        grid=grid,
        in_specs=[
            pl.BlockSpec((BT, ND), lambda i: (i, 0)),
            pl.BlockSpec((BT, F * D), lambda i: (i, 0)),
            pl.BlockSpec((BT, 1), lambda i: (i, 0)),
            full((ND, 1)),
            full((1, 1)),
            full((ND, H0)),
            full((F * D, H0)),
            full((1, H0)),
            full((H0, 512)),
            full((1, 512)),
            full((512, 256)),
            full((1, 256)),
            full((1, 1)),
            full((256, 1)),
            full((1, 1)),
        ],
        out_specs=pl.BlockSpec((BT, 1), lambda i: (i, 0)),
        out_shape=jax.ShapeDtypeStruct((B, 1), jnp.float32),
    )(numeric, emb2, fo_sum2, W_num, b_num, W0a, W0b, b0, W1, b1, W2, b2,
      Wfm, Wh, bout)


@jax.jit
def kernel(numeric, categorical, W_num, b_num, fo_tables, so_tables,
           W0, b0, W1, b1, W2, b2, Wout, bout):
    offs = (jnp.arange(F, dtype=jnp.int32) * V)[:, None]
    idx2d = (categorical.T + offs).reshape(IDX_ROWS, CH)
    so_flat = so_tables.reshape(F * V, D)
    fo_flat = fo_tables.reshape(F * V)

    emb2, fo_sum = _sc_gather(idx2d, so_flat, fo_flat)
    fo_sum2 = fo_sum.reshape(B, 1)

    W0a = W0[:ND]
    W0b = W0[ND:].astype(jnp.bfloat16)
    W1 = W1.astype(jnp.bfloat16)
    W2 = W2.astype(jnp.bfloat16)
    out = _tc_mlp(numeric, emb2, fo_sum2, W_num, b_num.reshape(1, 1),
                  W0a, W0b, b0.reshape(1, -1), W1, b1.reshape(1, -1),
                  W2, b2.reshape(1, -1), Wout[0:1], Wout[1:],
                  bout.reshape(1, 1))
    return out[:, 0]


# speedup vs baseline: 35.6412x; 1.1033x over previous
import functools

import jax
import jax.numpy as jnp
from jax import lax
from jax.experimental import pallas as pl
from jax.experimental.pallas import tpu as pltpu
from jax.experimental.pallas import tpu_sc as plsc

B = 16384
F = 26
V = 1001
D = 128
ND = 13
H0 = 1024

NC = 2
NS = 16
NW = NC * NS
CH = 128
NSPLIT = 2
BH = B // NSPLIT
BW = BH // NW
NCH = BW // CH
IDX_ROWS = F * BH // CH


def _sc_gather_body(idx_hbm, so_hbm, fo_hbm, emb_out, fo_out,
                    idx_v, rows_v, fo_v, acc_v, sem_g, sem_s):
    wid = lax.axis_index("s") * NC + lax.axis_index("c")
    b0 = wid * BW
    pltpu.sync_copy(fo_hbm, fo_v)
    for j in range(BW // 16):
        acc_v[pl.ds(j * 16, 16)] = jnp.zeros((16,), jnp.float32)

    row_base = wid * NCH

    def field_body(f, _):
        r0 = f * (BH // CH) + row_base
        pltpu.sync_copy(idx_hbm.at[pl.ds(r0, NCH)], idx_v)

        @pl.when(f > 0)
        def _():
            pltpu.make_async_copy(
                rows_v, emb_out.at[pl.ds(0, BW), pl.ds(0, D)], sem_s).wait()

        for c in range(NCH):
            pltpu.async_copy(so_hbm.at[idx_v.at[c]],
                             rows_v.at[pl.ds(c * CH, CH)], sem_g)
        for c in range(NCH):
            pltpu.make_async_copy(so_hbm.at[idx_v.at[c]],
                                  rows_v.at[pl.ds(c * CH, CH)], sem_g).wait()

        for c in range(NCH):
            for j in range(CH // 16):
                iv = idx_v[c, pl.ds(j * 16, 16)]
                sl = pl.ds(c * CH + j * 16, 16)
                acc_v[sl] = acc_v[sl] + plsc.load_gather(fo_v, [iv])

        pltpu.async_copy(
            rows_v, emb_out.at[pl.ds(b0, BW), pl.ds(f * D, D)], sem_s)
        return 0

    lax.fori_loop(0, F, field_body, 0)
    pltpu.make_async_copy(
        rows_v, emb_out.at[pl.ds(0, BW), pl.ds(0, D)], sem_s).wait()
    pltpu.sync_copy(acc_v, fo_out.at[pl.ds(b0, BW)])


def _sc_gather(idx2d, so_flat, fo_flat):
    mesh = plsc.VectorSubcoreMesh(core_axis_name="c", subcore_axis_name="s")
    return pl.kernel(
        _sc_gather_body,
        out_type=(
            jax.ShapeDtypeStruct((BH, F * D), jnp.float32),
            jax.ShapeDtypeStruct((BH,), jnp.float32),
        ),
        mesh=mesh,
        compiler_params=pltpu.CompilerParams(needs_layout_passes=False),
        scratch_types=[
            pltpu.VMEM((NCH, CH), jnp.int32),
            pltpu.VMEM((BW, D), jnp.float32),
            pltpu.VMEM((F * V,), jnp.float32),
            pltpu.VMEM((BW,), jnp.float32),
            pltpu.SemaphoreType.DMA,
            pltpu.SemaphoreType.DMA,
        ],
    )(idx2d, so_flat, fo_flat)


BT = 512


def _tc_body(num_ref, emb_ref, fo_ref, wnum_ref, bnum_ref,
             w0a_ref, w0b_ref, b0_ref, w1_ref, b1_ref, w2_ref, b2_ref,
             wfm_ref, wh_ref, bout_ref, out_ref):
    num = num_ref[...]
    flat = emb_ref[...]

    s = jnp.zeros((BT, D), jnp.float32)
    q = jnp.zeros((BT, D), jnp.float32)
    for f in range(F):
        x = flat[:, f * D:(f + 1) * D]
        s = s + x
        q = q + x * x
    sumsq = jnp.sum(s * s, axis=1, keepdims=True)
    sqsum = jnp.sum(q, axis=1, keepdims=True)
    fm2 = 0.5 * (sumsq - sqsum)

    fm1 = jnp.dot(num, wnum_ref[...], preferred_element_type=jnp.float32)
    fm1 = fm1 + bnum_ref[...] + fo_ref[...]
    fm = fm1 + fm2

    bf = jnp.bfloat16
    h = jnp.dot(num, w0a_ref[...], preferred_element_type=jnp.float32)
    h = h + jnp.dot(flat.astype(bf), w0b_ref[...],
                    preferred_element_type=jnp.float32)
    h = jnp.maximum(h + b0_ref[...], 0.0)
    h = jnp.maximum(jnp.dot(h.astype(bf), w1_ref[...],
                            preferred_element_type=jnp.float32)
                    + b1_ref[...], 0.0)
    h = jnp.maximum(jnp.dot(h.astype(bf), w2_ref[...],
                            preferred_element_type=jnp.float32)
                    + b2_ref[...], 0.0)

    total = fm * wfm_ref[0, 0] + jnp.dot(h, wh_ref[...],
                                         preferred_element_type=jnp.float32)
    total = total + bout_ref[...]
    out_ref[...] = jax.nn.sigmoid(total)


def _tc_mlp(numeric, emb2, fo_sum2, W_num, b_num, W0a, W0b, b0, W1, b1, W2, b2,
            Wfm, Wh, bout):
    grid = (BH // BT,)

    def full(shape):
        return pl.BlockSpec(shape, lambda *_: tuple(0 for _ in shape))

    return pl.pallas_call(
        _tc_body,
        grid=grid,
        in_specs=[
            pl.BlockSpec((BT, ND), lambda i: (i, 0)),
            pl.BlockSpec((BT, F * D), lambda i: (i, 0)),
            pl.BlockSpec((BT, 1), lambda i: (i, 0)),
            full((ND, 1)),
            full((1, 1)),
            full((ND, H0)),
            full((F * D, H0)),
            full((1, H0)),
            full((H0, 512)),
            full((1, 512)),
            full((512, 256)),
            full((1, 256)),
            full((1, 1)),
            full((256, 1)),
            full((1, 1)),
        ],
        out_specs=pl.BlockSpec((BT, 1), lambda i: (i, 0)),
        out_shape=jax.ShapeDtypeStruct((BH, 1), jnp.float32),
    )(numeric, emb2, fo_sum2, W_num, b_num, W0a, W0b, b0, W1, b1, W2, b2,
      Wfm, Wh, bout)


@jax.jit
def kernel(numeric, categorical, W_num, b_num, fo_tables, so_tables,
           W0, b0, W1, b1, W2, b2, Wout, bout):
    offs = (jnp.arange(F, dtype=jnp.int32) * V)[:, None]
    so_flat = so_tables.reshape(F * V, D)
    fo_flat = fo_tables.reshape(F * V)

    W0a = W0[:ND]
    W0b = W0[ND:].astype(jnp.bfloat16)
    W1 = W1.astype(jnp.bfloat16)
    W2 = W2.astype(jnp.bfloat16)

    gathered = []
    for h in range(NSPLIT):
        cat_h = categorical[h * BH:(h + 1) * BH]
        idx2d = (cat_h.T + offs).reshape(IDX_ROWS, CH)
        gathered.append(_sc_gather(idx2d, so_flat, fo_flat))

    outs = []
    for h in range(NSPLIT):
        emb2, fo_sum = gathered[h]
        out = _tc_mlp(numeric[h * BH:(h + 1) * BH], emb2,
                      fo_sum.reshape(BH, 1), W_num, b_num.reshape(1, 1),
                      W0a, W0b, b0.reshape(1, -1), W1, b1.reshape(1, -1),
                      W2, b2.reshape(1, -1), Wout[0:1], Wout[1:],
                      bout.reshape(1, 1))
        outs.append(out[:, 0])
    return jnp.concatenate(outs)


# speedup vs baseline: 37.4639x; 1.0511x over previous
import functools

import jax
import jax.numpy as jnp
from jax import lax
from jax.experimental import pallas as pl
from jax.experimental.pallas import tpu as pltpu
from jax.experimental.pallas import tpu_sc as plsc

B = 16384
F = 26
V = 1001
D = 128
ND = 13
H0 = 1024

NC = 2
NS = 16
NW = NC * NS
CH = 128
NSPLIT = 2
BH = B // NSPLIT
BW = BH // NW
NCH = BW // CH
NSTEP = F * NCH
NBUF = 5
LOOKAHEAD = 3


def _sc_gather_body(idx_hbm, so_hbm, fo_hbm, emb_out, fo_out,
                    idx_v, rows_v, fo_v, acc_v, sem_g, sem_s):
    wid = lax.axis_index("s") * NC + lax.axis_index("c")
    b0 = wid * BW
    pltpu.sync_copy(fo_hbm, fo_v)
    pltpu.sync_copy(idx_hbm.at[wid], idx_v)
    for j in range(BW // 16):
        acc_v[pl.ds(j * 16, 16)] = jnp.zeros((16,), jnp.float32)

    def store_dst(i):
        f, c = divmod(i, NCH)
        return emb_out.at[pl.ds(b0 + c * CH, CH), pl.ds(f * D, D)]

    def fire_gather(i):
        pltpu.async_copy(so_hbm.at[idx_v.at[i]], rows_v.at[i % NBUF], sem_g)

    for i in range(LOOKAHEAD):
        fire_gather(i)
    for i in range(NSTEP):
        if i + LOOKAHEAD < NSTEP:
            if i >= 2:
                pltpu.make_async_copy(
                    rows_v.at[(i - 2) % NBUF], store_dst(i - 2), sem_s).wait()
            fire_gather(i + LOOKAHEAD)
        pltpu.make_async_copy(
            so_hbm.at[idx_v.at[i]], rows_v.at[i % NBUF], sem_g).wait()
        c = i % NCH
        for j in range(CH // 16):
            iv = idx_v[i, pl.ds(j * 16, 16)]
            sl = pl.ds(c * CH + j * 16, 16)
            acc_v[sl] = acc_v[sl] + plsc.load_gather(fo_v, [iv])
        pltpu.async_copy(rows_v.at[i % NBUF], store_dst(i), sem_s)
    for i in range(NSTEP - NBUF, NSTEP):
        pltpu.make_async_copy(
            rows_v.at[i % NBUF], store_dst(i), sem_s).wait()
    pltpu.sync_copy(acc_v, fo_out.at[pl.ds(b0, BW)])


def _sc_gather(idx2d, so_flat, fo_flat):
    mesh = plsc.VectorSubcoreMesh(core_axis_name="c", subcore_axis_name="s")
    return pl.kernel(
        _sc_gather_body,
        out_type=(
            jax.ShapeDtypeStruct((BH, F * D), jnp.float32),
            jax.ShapeDtypeStruct((BH,), jnp.float32),
        ),
        mesh=mesh,
        compiler_params=pltpu.CompilerParams(needs_layout_passes=False),
        scratch_types=[
            pltpu.VMEM((NSTEP, CH), jnp.int32),
            pltpu.VMEM((NBUF, CH, D), jnp.float32),
            pltpu.VMEM((F * V,), jnp.float32),
            pltpu.VMEM((BW,), jnp.float32),
            pltpu.SemaphoreType.DMA,
            pltpu.SemaphoreType.DMA,
        ],
    )(idx2d, so_flat, fo_flat)


BT = 512


def _tc_body(num_ref, emb_ref, fo_ref, wnum_ref, bnum_ref,
             w0a_ref, w0b_ref, b0_ref, w1_ref, b1_ref, w2_ref, b2_ref,
             wfm_ref, wh_ref, bout_ref, out_ref):
    num = num_ref[...]
    flat = emb_ref[...]

    s = jnp.zeros((BT, D), jnp.float32)
    q = jnp.zeros((BT, D), jnp.float32)
    for f in range(F):
        x = flat[:, f * D:(f + 1) * D]
        s = s + x
        q = q + x * x
    sumsq = jnp.sum(s * s, axis=1, keepdims=True)
    sqsum = jnp.sum(q, axis=1, keepdims=True)
    fm2 = 0.5 * (sumsq - sqsum)

    fm1 = jnp.dot(num, wnum_ref[...], preferred_element_type=jnp.float32)
    fm1 = fm1 + bnum_ref[...] + fo_ref[...]
    fm = fm1 + fm2

    bf = jnp.bfloat16
    h = jnp.dot(num, w0a_ref[...], preferred_element_type=jnp.float32)
    h = h + jnp.dot(flat.astype(bf), w0b_ref[...],
                    preferred_element_type=jnp.float32)
    h = jnp.maximum(h + b0_ref[...], 0.0)
    h = jnp.maximum(jnp.dot(h.astype(bf), w1_ref[...],
                            preferred_element_type=jnp.float32)
                    + b1_ref[...], 0.0)
    h = jnp.maximum(jnp.dot(h.astype(bf), w2_ref[...],
                            preferred_element_type=jnp.float32)
                    + b2_ref[...], 0.0)

    total = fm * wfm_ref[0, 0] + jnp.dot(h, wh_ref[...],
                                         preferred_element_type=jnp.float32)
    total = total + bout_ref[...]
    out_ref[...] = jax.nn.sigmoid(total)


def _tc_mlp(numeric, emb2, fo_sum2, W_num, b_num, W0a, W0b, b0, W1, b1, W2, b2,
            Wfm, Wh, bout):
    grid = (BH // BT,)

    def full(shape):
        return pl.BlockSpec(shape, lambda *_: tuple(0 for _ in shape))

    return pl.pallas_call(
        _tc_body,
        grid=grid,
        in_specs=[
            pl.BlockSpec((BT, ND), lambda i: (i, 0)),
            pl.BlockSpec((BT, F * D), lambda i: (i, 0)),
            pl.BlockSpec((BT, 1), lambda i: (i, 0)),
            full((ND, 1)),
            full((1, 1)),
            full((ND, H0)),
            full((F * D, H0)),
            full((1, H0)),
            full((H0, 512)),
            full((1, 512)),
            full((512, 256)),
            full((1, 256)),
            full((1, 1)),
            full((256, 1)),
            full((1, 1)),
        ],
        out_specs=pl.BlockSpec((BT, 1), lambda i: (i, 0)),
        out_shape=jax.ShapeDtypeStruct((BH, 1), jnp.float32),
    )(numeric, emb2, fo_sum2, W_num, b_num, W0a, W0b, b0, W1, b1, W2, b2,
      Wfm, Wh, bout)


@jax.jit
def kernel(numeric, categorical, W_num, b_num, fo_tables, so_tables,
           W0, b0, W1, b1, W2, b2, Wout, bout):
    offs = (jnp.arange(F, dtype=jnp.int32) * V)[:, None]
    so_flat = so_tables.reshape(F * V, D)
    fo_flat = fo_tables.reshape(F * V)

    W0a = W0[:ND]
    W0b = W0[ND:].astype(jnp.bfloat16)
    W1 = W1.astype(jnp.bfloat16)
    W2 = W2.astype(jnp.bfloat16)

    gathered = []
    for h in range(NSPLIT):
        cat_h = categorical[h * BH:(h + 1) * BH]
        idx_fw = (cat_h.T + offs).reshape(F, NW, BW)
        idx_w = jnp.transpose(idx_fw, (1, 0, 2)).reshape(NW, NSTEP, CH)
        gathered.append(_sc_gather(idx_w, so_flat, fo_flat))

    outs = []
    for h in range(NSPLIT):
        emb2, fo_sum = gathered[h]
        out = _tc_mlp(numeric[h * BH:(h + 1) * BH], emb2,
                      fo_sum.reshape(BH, 1), W_num, b_num.reshape(1, 1),
                      W0a, W0b, b0.reshape(1, -1), W1, b1.reshape(1, -1),
                      W2, b2.reshape(1, -1), Wout[0:1], Wout[1:],
                      bout.reshape(1, 1))
        outs.append(out[:, 0])
    return jnp.concatenate(outs)
